# baseline (speedup 1.0000x reference)
import functools

import jax
import jax.numpy as jnp
from jax import lax
from jax.experimental import pallas as pl
from jax.experimental.pallas import tpu as pltpu
from jax.experimental.pallas import tpu_sc as plsc

N_ITEMS = 10000
N_PERSONS = 100000
N_GRADES = 5
N_RESP = 1048576

NC, NS, L = 2, 16, 16
NW = NC * NS
PER_W = N_RESP // NW
CH = 2048
N_CH = PER_W // CH

_LOG2PI = 1.8378770664093453


def _softplus(x):
    return jnp.maximum(x, 0.0) + jnp.log1p(jnp.exp(-jnp.abs(x)))


def _prep_body(a_ref, bb_ref, d0_ref, d1_ref, d2_ref, t_ref,
               a_out, b0_out, b1_out, b2_out, b3_out, prior_out):
    a = _softplus(a_ref[...])
    b0 = bb_ref[...]
    b1 = b0 + _softplus(d0_ref[...])
    b2 = b1 + _softplus(d1_ref[...])
    b3 = b2 + _softplus(d2_ref[...])
    a_out[...] = a
    b0_out[...] = b0
    b1_out[...] = b1
    b2_out[...] = b2
    b3_out[...] = b3
    n_elem = N_ITEMS + N_ITEMS * (N_GRADES - 1) + N_PERSONS
    sq = (jnp.sum(a * a) + jnp.sum(b0 * b0) + jnp.sum(b1 * b1)
          + jnp.sum(b2 * b2) + jnp.sum(b3 * b3)
          + jnp.sum(t_ref[...] * t_ref[...]))
    prior_out[0, 0] = -0.5 * _LOG2PI * n_elem - 0.5 * sq


def _prep(a_, b_base_, b_diff_, t):
    vec = jax.ShapeDtypeStruct((N_ITEMS,), jnp.float32)
    return pl.pallas_call(
        _prep_body,
        out_shape=(vec, vec, vec, vec, vec,
                   jax.ShapeDtypeStruct((1, 1), jnp.float32)),
        out_specs=(pl.BlockSpec(), pl.BlockSpec(), pl.BlockSpec(),
                   pl.BlockSpec(), pl.BlockSpec(),
                   pl.BlockSpec(memory_space=pltpu.SMEM)),
    )(a_, b_base_.reshape(N_ITEMS),
      b_diff_[:, 0], b_diff_[:, 1], b_diff_[:, 2], t)


def _sc_body(item_h, person_h, resp_h, a_h, b0_h, b1_h, b2_h, b3_h, t2_h, p_h,
             a_v, b_v, t2_v, it0, it1, pe0, pe1, rs0, rs1, pb0, pb1,
             si0, si1, sp0, sp1):
    wid = lax.axis_index("s") * NC + lax.axis_index("c")
    pltpu.sync_copy(a_h, a_v)
    for k, bk in enumerate((b0_h, b1_h, b2_h, b3_h)):
        pltpu.sync_copy(bk, b_v.at[pl.ds(k * N_ITEMS, N_ITEMS)])
    pltpu.sync_copy(t2_h, t2_v)
    base = wid * PER_W

    ibufs = ((it0, pe0, rs0), (it1, pe1, rs1))
    pbufs = (pb0, pb1)
    isems = (si0, si1)
    psems = (sp0, sp1)

    def fire_idx(ci, b):
        off = base + lax.rem(ci, N_CH) * CH
        for src, dst in zip((item_h, person_h, resp_h), ibufs[b]):
            pltpu.async_copy(src.at[pl.ds(off, CH)], dst, isems[b])

    fire_idx(jnp.int32(0), 0)
    fire_idx(jnp.int32(1), 1)

    def pair(k, carry):
        for b in range(2):
            ci = 2 * k + b
            (it_v, pe_v, rs_v), pb = ibufs[b], pbufs[b]
            for src, dst in zip((item_h, person_h, resp_h), ibufs[b]):
                pltpu.make_async_copy(
                    src.at[pl.ds(0, CH)], dst, isems[b]).wait()
            @pl.when(k >= 1)
            def _():
                pltpu.make_async_copy(
                    pb, p_h.at[pl.ds(0, CH)], psems[b]).wait()

            @plsc.parallel_loop(0, CH // L, unroll=8)
            def step(s):
                sl = pl.ds(s * L, L)
                it = it_v[sl]
                pe = pe_v[sl]
                rs = rs_v[sl]
                tw = plsc.load_gather(t2_v, [pe >> 1])
                odd = (pe & 1) == 1
                bits = jnp.where(odd, tw & jnp.int32(-65536), tw << 16)
                tv = plsc.bitcast(bits, jnp.float32)
                av = plsc.load_gather(a_v, [it])
                bi = it + rs * N_ITEMS
                bu = plsc.load_gather(b_v, [jnp.maximum(bi - 2 * N_ITEMS, 0)])
                bl = plsc.load_gather(
                    b_v, [jnp.minimum(bi - N_ITEMS, N_ITEMS * 4 - 1)])
                su = 1.0 / (1.0 + jnp.exp(av * (bu - tv)))
                slo = 1.0 / (1.0 + jnp.exp(av * (bl - tv)))
                upper = jnp.where(rs == 1, 1.0, su)
                lower = jnp.where(rs == N_GRADES, 0.0, slo)
                pb[pl.ds(s * L, L)] = jnp.clip(upper - lower, 1e-12, 1.0)

            pltpu.async_copy(pb, p_h.at[pl.ds(base + ci * CH, CH)], psems[b])
            fire_idx(ci + 2, b)
        return carry

    lax.fori_loop(0, N_CH // 2, pair, 0)

    for b in range(2):
        pltpu.make_async_copy(
            pbufs[b], p_h.at[pl.ds(0, CH)], psems[b]).wait()
        for src, dst in zip((item_h, person_h, resp_h), ibufs[b]):
            pltpu.make_async_copy(
                src.at[pl.ds(0, CH)], dst, isems[b]).wait()


def _sc_gather(item, person, resp, a, b0, b1, b2, b3, t2):
    mesh = plsc.VectorSubcoreMesh(
        core_axis_name="c", subcore_axis_name="s",
        num_cores=NC, num_subcores=NS)
    f = functools.partial(
        pl.kernel,
        out_type=jax.ShapeDtypeStruct((N_RESP,), jnp.float32),
        mesh=mesh,
        scratch_types=[
            pltpu.VMEM((N_ITEMS,), jnp.float32),
            pltpu.VMEM((N_ITEMS * 4,), jnp.float32),
            pltpu.VMEM((N_PERSONS // 2,), jnp.int32),
            pltpu.VMEM((CH,), jnp.int32),
            pltpu.VMEM((CH,), jnp.int32),
            pltpu.VMEM((CH,), jnp.int32),
            pltpu.VMEM((CH,), jnp.int32),
            pltpu.VMEM((CH,), jnp.int32),
            pltpu.VMEM((CH,), jnp.int32),
            pltpu.VMEM((CH,), jnp.float32),
            pltpu.VMEM((CH,), jnp.float32),
            pltpu.SemaphoreType.DMA,
            pltpu.SemaphoreType.DMA,
            pltpu.SemaphoreType.DMA,
            pltpu.SemaphoreType.DMA,
        ],
        compiler_params=pltpu.CompilerParams(needs_layout_passes=False),
    )(_sc_body)
    return f(item, person, resp, a, b0, b1, b2, b3, t2)


def _reduce_body(p_ref, prior_ref, out_ref):
    ll = jnp.sum(jnp.log(p_ref[...]))
    out_ref[0, 0] = -(ll + prior_ref[0, 0])


def _reduce(p2d, prior):
    return pl.pallas_call(
        _reduce_body,
        out_shape=jax.ShapeDtypeStruct((1, 1), jnp.float32),
        in_specs=(
            pl.BlockSpec(),
            pl.BlockSpec(memory_space=pltpu.SMEM),
        ),
        out_specs=pl.BlockSpec(memory_space=pltpu.SMEM),
    )(p2d, prior)


def kernel(indices, a_, b_base_, b_diff_, t):
    item = indices[:, 0]
    person = indices[:, 1]
    resp = indices[:, 2]
    t2 = jax.lax.bitcast_convert_type(
        t.astype(jnp.bfloat16).reshape(N_PERSONS // 2, 2), jnp.int32)
    a, b0, b1, b2, b3, prior = _prep(a_, b_base_, b_diff_, t)
    p = _sc_gather(item, person, resp, a, b0, b1, b2, b3, t2)
    out = _reduce(p.reshape(N_RESP // 128, 128), prior)
    return out.reshape(())


# speedup vs baseline: 1.4701x; 1.4701x over previous
import functools

import jax
import jax.numpy as jnp
from jax import lax
from jax.experimental import pallas as pl
from jax.experimental.pallas import tpu as pltpu
from jax.experimental.pallas import tpu_sc as plsc

N_ITEMS = 10000
N_PERSONS = 100000
N_GRADES = 5
N_RESP = 1048576

NC, NS, L = 2, 16, 16
NW = NC * NS
PER_W = N_RESP // NW
CH = 2048
N_CH = PER_W // CH

_LOG2PI = 1.8378770664093453


def _softplus(x):
    return jnp.maximum(x, 0.0) + jnp.log1p(jnp.exp(-jnp.abs(x)))


_HALF = N_PERSONS // 2


def _bf16_bits(x):
    return lax.bitcast_convert_type(
        x.astype(jnp.bfloat16), jnp.uint16).astype(jnp.int32)


def _prep_body(a_ref, bb_ref, d0_ref, d1_ref, d2_ref, t_ref,
               a_out, b0_out, b1_out, b2_out, b3_out, t2_out, prior_out):
    a = _softplus(a_ref[...])
    b0 = bb_ref[...]
    b1 = b0 + _softplus(d0_ref[...])
    b2 = b1 + _softplus(d1_ref[...])
    b3 = b2 + _softplus(d2_ref[...])
    a_out[...] = a
    b0_out[...] = b0
    b1_out[...] = b1
    b2_out[...] = b2
    b3_out[...] = b3
    t = t_ref[...]
    t2_out[...] = _bf16_bits(t[:_HALF]) | (_bf16_bits(t[_HALF:]) << 16)
    n_elem = N_ITEMS + N_ITEMS * (N_GRADES - 1) + N_PERSONS
    sq = (jnp.sum(a * a) + jnp.sum(b0 * b0) + jnp.sum(b1 * b1)
          + jnp.sum(b2 * b2) + jnp.sum(b3 * b3) + jnp.sum(t * t))
    prior_out[0, 0] = -0.5 * _LOG2PI * n_elem - 0.5 * sq


def _prep(a_, b_base_, b_diff_, t):
    vec = jax.ShapeDtypeStruct((N_ITEMS,), jnp.float32)
    return pl.pallas_call(
        _prep_body,
        out_shape=(vec, vec, vec, vec, vec,
                   jax.ShapeDtypeStruct((_HALF,), jnp.int32),
                   jax.ShapeDtypeStruct((1, 1), jnp.float32)),
        out_specs=(pl.BlockSpec(), pl.BlockSpec(), pl.BlockSpec(),
                   pl.BlockSpec(), pl.BlockSpec(), pl.BlockSpec(),
                   pl.BlockSpec(memory_space=pltpu.SMEM)),
    )(a_, b_base_.reshape(N_ITEMS),
      b_diff_[:, 0], b_diff_[:, 1], b_diff_[:, 2], t)


def _sc_body(item_h, person_h, resp_h, a_h, b0_h, b1_h, b2_h, b3_h, t2_h, p_h,
             a_v, b_v, t2_v, it0, it1, pe0, pe1, rs0, rs1, pb0, pb1,
             si0, si1, sp0, sp1):
    wid = lax.axis_index("s") * NC + lax.axis_index("c")
    pltpu.sync_copy(a_h, a_v)
    for k, bk in enumerate((b0_h, b1_h, b2_h, b3_h)):
        pltpu.sync_copy(bk, b_v.at[pl.ds(k * N_ITEMS, N_ITEMS)])
    pltpu.sync_copy(t2_h, t2_v)
    base = wid * PER_W

    ibufs = ((it0, pe0, rs0), (it1, pe1, rs1))
    pbufs = (pb0, pb1)
    isems = (si0, si1)
    psems = (sp0, sp1)

    def fire_idx(ci, b):
        off = base + lax.rem(ci, N_CH) * CH
        for src, dst in zip((item_h, person_h, resp_h), ibufs[b]):
            pltpu.async_copy(src.at[pl.ds(off, CH)], dst, isems[b])

    fire_idx(jnp.int32(0), 0)
    fire_idx(jnp.int32(1), 1)

    def pair(k, carry):
        for b in range(2):
            ci = 2 * k + b
            (it_v, pe_v, rs_v), pb = ibufs[b], pbufs[b]
            for src, dst in zip((item_h, person_h, resp_h), ibufs[b]):
                pltpu.make_async_copy(
                    src.at[pl.ds(0, CH)], dst, isems[b]).wait()
            @pl.when(k >= 1)
            def _():
                pltpu.make_async_copy(
                    pb, p_h.at[pl.ds(0, CH)], psems[b]).wait()

            @plsc.parallel_loop(0, CH // L, unroll=4)
            def step(s):
                sl = pl.ds(s * L, L)
                it = it_v[sl]
                pe = pe_v[sl]
                rs = rs_v[sl]
                hi = pe >= _HALF
                tw = plsc.load_gather(
                    t2_v, [pe - jnp.where(hi, _HALF, 0)])
                bits = jnp.where(hi, tw & jnp.int32(-65536), tw << 16)
                tv = plsc.bitcast(bits, jnp.float32)
                av = plsc.load_gather(a_v, [it])
                bi = it + rs * N_ITEMS
                bu = plsc.load_gather(b_v, [jnp.maximum(bi - 2 * N_ITEMS, 0)])
                bl = plsc.load_gather(
                    b_v, [jnp.minimum(bi - N_ITEMS, N_ITEMS * 4 - 1)])
                su = 1.0 / (1.0 + jnp.exp(av * (bu - tv)))
                slo = 1.0 / (1.0 + jnp.exp(av * (bl - tv)))
                upper = jnp.where(rs == 1, 1.0, su)
                lower = jnp.where(rs == N_GRADES, 0.0, slo)
                pb[pl.ds(s * L, L)] = jnp.clip(upper - lower, 1e-12, 1.0)

            pltpu.async_copy(pb, p_h.at[pl.ds(base + ci * CH, CH)], psems[b])
            fire_idx(ci + 2, b)
        return carry

    lax.fori_loop(0, N_CH // 2, pair, 0)

    for b in range(2):
        pltpu.make_async_copy(
            pbufs[b], p_h.at[pl.ds(0, CH)], psems[b]).wait()
        for src, dst in zip((item_h, person_h, resp_h), ibufs[b]):
            pltpu.make_async_copy(
                src.at[pl.ds(0, CH)], dst, isems[b]).wait()


def _sc_gather(item, person, resp, a, b0, b1, b2, b3, t2):
    mesh = plsc.VectorSubcoreMesh(
        core_axis_name="c", subcore_axis_name="s",
        num_cores=NC, num_subcores=NS)
    f = functools.partial(
        pl.kernel,
        out_type=jax.ShapeDtypeStruct((N_RESP,), jnp.float32),
        mesh=mesh,
        scratch_types=[
            pltpu.VMEM((N_ITEMS,), jnp.float32),
            pltpu.VMEM((N_ITEMS * 4,), jnp.float32),
            pltpu.VMEM((N_PERSONS // 2,), jnp.int32),
            pltpu.VMEM((CH,), jnp.int32),
            pltpu.VMEM((CH,), jnp.int32),
            pltpu.VMEM((CH,), jnp.int32),
            pltpu.VMEM((CH,), jnp.int32),
            pltpu.VMEM((CH,), jnp.int32),
            pltpu.VMEM((CH,), jnp.int32),
            pltpu.VMEM((CH,), jnp.float32),
            pltpu.VMEM((CH,), jnp.float32),
            pltpu.SemaphoreType.DMA,
            pltpu.SemaphoreType.DMA,
            pltpu.SemaphoreType.DMA,
            pltpu.SemaphoreType.DMA,
        ],
        compiler_params=pltpu.CompilerParams(needs_layout_passes=False),
    )(_sc_body)
    return f(item, person, resp, a, b0, b1, b2, b3, t2)


_RG = 16
_RROWS = N_RESP // 128 // _RG


def _reduce_body(p_ref, prior_ref, out_ref, acc_ref):
    i = pl.program_id(0)

    @pl.when(i == 0)
    def _():
        acc_ref[0] = 0.0

    acc_ref[0] += jnp.sum(jnp.log(p_ref[...]))

    @pl.when(i == _RG - 1)
    def _():
        out_ref[0, 0] = -(acc_ref[0] + prior_ref[0, 0])


def _reduce(p2d, prior):
    return pl.pallas_call(
        _reduce_body,
        grid=(_RG,),
        out_shape=jax.ShapeDtypeStruct((1, 1), jnp.float32),
        in_specs=(
            pl.BlockSpec((_RROWS, 128), lambda i: (i, 0)),
            pl.BlockSpec(memory_space=pltpu.SMEM),
        ),
        out_specs=pl.BlockSpec(memory_space=pltpu.SMEM),
        scratch_shapes=[pltpu.SMEM((1,), jnp.float32)],
    )(p2d, prior)


def kernel(indices, a_, b_base_, b_diff_, t):
    item = indices[:, 0]
    person = indices[:, 1]
    resp = indices[:, 2]
    a, b0, b1, b2, b3, t2, prior = _prep(a_, b_base_, b_diff_, t)
    p = _sc_gather(item, person, resp, a, b0, b1, b2, b3, t2)
    out = _reduce(p.reshape(N_RESP // 128, 128), prior)
    return out.reshape(())
